# Initial kernel scaffold; baseline (speedup 1.0000x reference)
import jax
import jax.numpy as jnp
from jax.experimental import pallas as pl


def kernel(x, edge_index, W_gc1, b_gc1, g1, be1, W_gc2, b_gc2, g2, be2, W_fc1, b_fc1, W_fc2, b_fc2):
    raise NotImplementedError("write your pallas kernel here")


# speedup vs baseline: 23.3698x; 23.3698x over previous
import functools

import jax
import jax.numpy as jnp
from jax import lax
from jax.experimental import pallas as pl
from jax.experimental.pallas import tpu as pltpu
from jax.experimental.pallas import tpu_sc as plsc

N = 10000
NPAD = 10240
E = 320000
D_IN = 128
HID = 64
LAT = 32

NC = 2
NS = 16
NW = NC * NS
EPW = E // NW
CH = 80
NCHUNK = EPW // CH
RPT = NPAD // NS

_MESH = plsc.VectorSubcoreMesh(core_axis_name="c", subcore_axis_name="s")


def _make_sc_scatter(width):
    @functools.partial(
        pl.kernel,
        out_type=jax.ShapeDtypeStruct((NC, NPAD, width), jnp.float32),
        mesh=_MESH,
        compiler_params=pltpu.CompilerParams(use_tc_tiling_on_sc=False),
        scratch_types=[
            pltpu.VMEM((NCHUNK, CH), jnp.int32),
            pltpu.VMEM((NCHUNK, CH), jnp.int32),
            pltpu.VMEM((CH, width), jnp.float32),
            pltpu.VMEM_SHARED((NPAD, width), jnp.float32),
            pltpu.SemaphoreType.DMA,
        ],
    )
    def sc_scatter(table_hbm, src_hbm, dst_hbm, zero_hbm, out_hbm,
                   src_v, dst_v, rows_v, accum, gsem):
        c = lax.axis_index("c")
        s = lax.axis_index("s")
        wid = c * NS + s
        pltpu.sync_copy(zero_hbm.at[pl.ds(s * RPT, RPT)],
                        accum.at[pl.ds(s * RPT, RPT)])
        pltpu.sync_copy(src_hbm.at[wid], src_v)
        pltpu.sync_copy(dst_hbm.at[wid], dst_v)
        plsc.subcore_barrier()

        def body(j, _):
            pltpu.async_copy(table_hbm.at[src_v.at[j]], rows_v, gsem).wait()
            pltpu.sync_copy(rows_v, accum.at[dst_v.at[j]], add=True)
            return ()

        lax.fori_loop(0, NCHUNK, body, (), unroll=False)
        plsc.subcore_barrier()
        pltpu.sync_copy(accum.at[pl.ds(s * RPT, RPT)],
                        out_hbm.at[c, pl.ds(s * RPT, RPT)])

    return sc_scatter


def _make_sc_degree():
    @functools.partial(
        pl.kernel,
        out_type=jax.ShapeDtypeStruct((NC, NPAD, 16), jnp.float32),
        mesh=_MESH,
        compiler_params=pltpu.CompilerParams(use_tc_tiling_on_sc=False),
        scratch_types=[
            pltpu.VMEM((NCHUNK, CH), jnp.int32),
            pltpu.VMEM((CH, 16), jnp.float32),
            pltpu.VMEM_SHARED((NPAD, 16), jnp.float32),
        ],
    )
    def sc_degree(dst_hbm, zero_hbm, out_hbm, dst_v, ones_v, accum):
        c = lax.axis_index("c")
        s = lax.axis_index("s")
        wid = c * NS + s
        pltpu.sync_copy(zero_hbm.at[pl.ds(s * RPT, RPT)],
                        accum.at[pl.ds(s * RPT, RPT)])
        pltpu.sync_copy(dst_hbm.at[wid], dst_v)

        def fill(i, _):
            ones_v[pl.ds(i * 16, 16), :] = jnp.ones((16, 16), jnp.float32)
            return ()

        lax.fori_loop(0, CH // 16, fill, (), unroll=True)
        plsc.subcore_barrier()

        def body(j, _):
            pltpu.sync_copy(ones_v, accum.at[dst_v.at[j]], add=True)
            return ()

        lax.fori_loop(0, NCHUNK, body, (), unroll=False)
        plsc.subcore_barrier()
        pltpu.sync_copy(accum.at[pl.ds(s * RPT, RPT)],
                        out_hbm.at[c, pl.ds(s * RPT, RPT)])

    return sc_degree


_sc_scatter64 = _make_sc_scatter(HID)
_sc_scatter32 = _make_sc_scatter(LAT)
_sc_degree = _make_sc_degree()

_BM = 640


def _tc_h1_body(x_ref, w_ref, dis_ref, o_ref):
    h = jnp.dot(x_ref[...], w_ref[...], preferred_element_type=jnp.float32)
    o_ref[...] = h * dis_ref[...]


def _tc_mid_body(p_ref, hs1_ref, dis_ref, b1_ref, g1_ref, be1_ref, w2_ref,
                 o_ref):
    dis = dis_ref[...]
    t = (p_ref[0] + p_ref[1] + hs1_ref[...]) * dis + b1_ref[...]
    t = jnp.maximum(t, 0.0)
    mu = jnp.mean(t, axis=-1, keepdims=True)
    var = jnp.mean((t - mu) ** 2, axis=-1, keepdims=True)
    t = (t - mu) * lax.rsqrt(var + 1e-5) * g1_ref[...] + be1_ref[...]
    o_ref[...] = jnp.dot(t, w2_ref[...],
                         preferred_element_type=jnp.float32) * dis


def _tc_final_body(q_ref, hs2_ref, dis_ref, b2_ref, g2_ref, be2_ref,
                   wf1_ref, bf1_ref, wf2_ref, bf2_ref, lat_ref, rec_ref):
    dis = dis_ref[...]
    t = (q_ref[0] + q_ref[1] + hs2_ref[...]) * dis + b2_ref[...]
    t = jnp.maximum(t, 0.0)
    mu = jnp.mean(t, axis=-1, keepdims=True)
    var = jnp.mean((t - mu) ** 2, axis=-1, keepdims=True)
    t = (t - mu) * lax.rsqrt(var + 1e-5) * g2_ref[...] + be2_ref[...]
    lat_ref[...] = t
    d = jnp.dot(t, wf1_ref[...], preferred_element_type=jnp.float32)
    d = jnp.maximum(d + bf1_ref[...], 0.0)
    rec_ref[...] = jnp.dot(d, wf2_ref[...],
                           preferred_element_type=jnp.float32) + bf2_ref[...]


def _row_spec(width):
    return pl.BlockSpec((_BM, width), lambda i: (i, 0))


def _rep_spec(shape):
    nd = len(shape)
    return pl.BlockSpec(shape, lambda i: (0,) * nd)


_tc_h1 = pl.pallas_call(
    _tc_h1_body,
    grid=(NPAD // _BM,),
    in_specs=[_row_spec(D_IN), _rep_spec((D_IN, HID)), _row_spec(1)],
    out_specs=_row_spec(HID),
    out_shape=jax.ShapeDtypeStruct((NPAD, HID), jnp.float32),
)

_tc_mid = pl.pallas_call(
    _tc_mid_body,
    grid=(NPAD // _BM,),
    in_specs=[
        pl.BlockSpec((NC, _BM, HID), lambda i: (0, i, 0)),
        _row_spec(HID), _row_spec(1),
        _rep_spec((1, HID)), _rep_spec((1, HID)), _rep_spec((1, HID)),
        _rep_spec((HID, LAT)),
    ],
    out_specs=_row_spec(LAT),
    out_shape=jax.ShapeDtypeStruct((NPAD, LAT), jnp.float32),
)

_tc_final = pl.pallas_call(
    _tc_final_body,
    grid=(NPAD // _BM,),
    in_specs=[
        pl.BlockSpec((NC, _BM, LAT), lambda i: (0, i, 0)),
        _row_spec(LAT), _row_spec(1),
        _rep_spec((1, LAT)), _rep_spec((1, LAT)), _rep_spec((1, LAT)),
        _rep_spec((LAT, HID)), _rep_spec((1, HID)),
        _rep_spec((HID, D_IN)), _rep_spec((1, D_IN)),
    ],
    out_specs=[_row_spec(LAT), _row_spec(D_IN)],
    out_shape=[
        jax.ShapeDtypeStruct((NPAD, LAT), jnp.float32),
        jax.ShapeDtypeStruct((NPAD, D_IN), jnp.float32),
    ],
)


def kernel(x, edge_index, W_gc1, b_gc1, g1, be1, W_gc2, b_gc2, g2, be2,
           W_fc1, b_fc1, W_fc2, b_fc2):
    src = edge_index[0].reshape(NW, NCHUNK, CH)
    dst = edge_index[1].reshape(NW, NCHUNK, CH)
    x_pad = jnp.pad(x, ((0, NPAD - N), (0, 0)))
    z16 = jnp.zeros((NPAD, 16), jnp.float32)
    z64 = jnp.zeros((NPAD, HID), jnp.float32)
    z32 = jnp.zeros((NPAD, LAT), jnp.float32)

    degp = _sc_degree(dst, z16)
    deg = degp[0, :, 0] + degp[1, :, 0] + 1.0
    dis = lax.rsqrt(deg)[:, None]

    hs1 = _tc_h1(x_pad, W_gc1, dis)
    p1 = _sc_scatter64(hs1, src, dst, z64)
    hs2 = _tc_mid(p1, hs1, dis, b_gc1[None, :], g1[None, :], be1[None, :],
                  W_gc2)
    p2 = _sc_scatter32(hs2, src, dst, z32)
    latent, recon = _tc_final(p2, hs2, dis, b_gc2[None, :], g2[None, :],
                              be2[None, :], W_fc1, b_fc1[None, :], W_fc2,
                              b_fc2[None, :])
    return (latent[:N], recon[:N])
